# baseline (speedup 1.0000x reference)
import jax
import jax.numpy as jnp
from jax import lax
from jax.experimental import pallas as pl
from jax.experimental.pallas import tpu as pltpu
from jax.experimental.pallas import tpu_sc as plsc

NUM_ENT = 1000000
EDIM = 32
HDIM = 32
B = 16384

CB = 16384
SLAB = CB // 4
NCB = -(-NUM_ENT // CB)
NPR = NCB * SLAB
W4 = 128

NC = 2
NS = 16
L = 16
NW = NC * NS
BPW = B // NW
CH = 128
NCHUNK = BPW // CH


def _score_body(head_hbm, rel_hbm, tail_hbm, er_hbm, ei_hbm, rel_tab_hbm,
                out_hbm,
                hidx, tidx, relv, hgrp, tgrp, pkv, hr, hi, tr, ti, rtab, sco,
                sem):
    wid = lax.axis_index("s") * NC + lax.axis_index("c")
    base = wid * BPW
    pltpu.sync_copy(head_hbm.at[pl.ds(base, BPW)], hidx)
    pltpu.sync_copy(tail_hbm.at[pl.ds(base, BPW)], tidx)
    pltpu.sync_copy(rel_hbm.at[pl.ds(base, BPW)], relv)
    pltpu.sync_copy(rel_tab_hbm, rtab)

    def mkgrp(s, _):
        hc = hidx[pl.ds(s * L, L)]
        tc = tidx[pl.ds(s * L, L)]
        rc = relv[pl.ds(s * L, L)]
        hgrp[pl.ds(s * L, L)] = (
            lax.shift_left(lax.shift_right_logical(hc, 14), 12)
            | (hc & 4095))
        tgrp[pl.ds(s * L, L)] = (
            lax.shift_left(lax.shift_right_logical(tc, 14), 12)
            | (tc & 4095))
        pkv[pl.ds(s * L, L)] = (
            (lax.shift_right_logical(hc, 12) & 3)
            | lax.shift_left(lax.shift_right_logical(tc, 12) & 3, 2)
            | lax.shift_left(rc, 4))
        return _
    lax.fori_loop(0, BPW // L, mkgrp, 0)

    rr0 = [rtab[pl.ds(k * L, L)] for k in range(2)]
    ri0 = [rtab[pl.ds(EDIM + k * L, L)] for k in range(2)]
    rr1 = [rtab[pl.ds(2 * EDIM + k * L, L)] for k in range(2)]
    ri1 = [rtab[pl.ds(3 * EDIM + k * L, L)] for k in range(2)]
    lane = lax.broadcasted_iota(jnp.int32, (L,), 0)

    _gd = lax.GatherDimensionNumbers(
        offset_dims=(), collapsed_slice_dims=(0,), start_index_map=(0,))

    def vperm(v, idx):
        return lax.gather(v, idx[:, None], _gd, (1,),
                          mode=lax.GatherScatterMode.PROMISE_IN_BOUNDS)

    perm_idx = [lane ^ sh for sh in (8, 4, 2, 1)]

    def lane_sum(v):
        for idx in perm_idx:
            v = v + vperm(v, idx)
        return v

    jconst = [jnp.full((L,), j, jnp.int32) for j in range(L)]

    for c in range(NCHUNK):
        s = c * CH
        cps = [
            pltpu.async_copy(er_hbm.at[hgrp.at[pl.ds(s, CH)]], hr, sem),
            pltpu.async_copy(ei_hbm.at[hgrp.at[pl.ds(s, CH)]], hi, sem),
            pltpu.async_copy(er_hbm.at[tgrp.at[pl.ds(s, CH)]], tr, sem),
            pltpu.async_copy(ei_hbm.at[tgrp.at[pl.ds(s, CH)]], ti, sem),
        ]
        for cp in cps:
            cp.wait()

        def group(g, carry):
            pkc = pkv[pl.ds(s + g * L, L)]
            res = jnp.zeros((L,), jnp.float32)
            for j in range(L):
                r0 = g * L + j
                w = pkc[j]
                hoff = (w & 3) * EDIM
                toff = ((w >> 2) & 3) * EDIM
                rsel = w >> 4
                acc = jnp.zeros((L,), jnp.float32)
                for k in range(2):
                    hrk = hr[r0, pl.ds(hoff + k * L, L)]
                    hik = hi[r0, pl.ds(hoff + k * L, L)]
                    trk = tr[r0, pl.ds(toff + k * L, L)]
                    tik = ti[r0, pl.ds(toff + k * L, L)]
                    rrk = jnp.where(rsel == 0, rr0[k], rr1[k])
                    rik = jnp.where(rsel == 0, ri0[k], ri1[k])
                    u = hrk * rrk - hik * rik
                    v = hik * rrk + hrk * rik
                    acc = acc + trk * u + tik * v
                ssum = lane_sum(acc)
                res = jnp.where(lane == j, ssum, res)
            sco[pl.ds(s + g * L, L)] = res
            return carry

        lax.fori_loop(0, CH // L, group, 0)

    pltpu.sync_copy(sco, out_hbm.at[pl.ds(base, BPW)])


def _score_sc(head, relation, tail, er4, ei4, rel_tab):
    mesh = plsc.VectorSubcoreMesh(core_axis_name="c", subcore_axis_name="s",
                                  num_cores=NC, num_subcores=NS)
    fn = pl.kernel(
        _score_body,
        out_type=jax.ShapeDtypeStruct((B,), jnp.float32),
        mesh=mesh,
        scratch_types=[
            pltpu.VMEM((BPW,), jnp.int32),
            pltpu.VMEM((BPW,), jnp.int32),
            pltpu.VMEM((BPW,), jnp.int32),
            pltpu.VMEM((BPW,), jnp.int32),
            pltpu.VMEM((BPW,), jnp.int32),
            pltpu.VMEM((BPW,), jnp.int32),
            pltpu.VMEM((CH, W4), jnp.float32),
            pltpu.VMEM((CH, W4), jnp.float32),
            pltpu.VMEM((CH, W4), jnp.float32),
            pltpu.VMEM((CH, W4), jnp.float32),
            pltpu.VMEM((4 * EDIM,), jnp.float32),
            pltpu.VMEM((BPW,), jnp.float32),
            pltpu.SemaphoreType.DMA,
        ],
    )
    return fn(head, relation, tail, er4, ei4, rel_tab)


def _fc_body(ert_ref, eit_ref, w_ref, b_ref, nft_ref, er4_ref, ei4_ref):
    er = ert_ref[...]
    ei = eit_ref[...]
    x = er + ei
    y = jnp.dot(w_ref[...], x, preferred_element_type=jnp.float32)
    nft_ref[...] = jnp.maximum(y + b_ref[...], 0.0)
    er4_ref[...] = jnp.concatenate(
        [er[:, p * SLAB:(p + 1) * SLAB].T for p in range(4)], axis=1)
    ei4_ref[...] = jnp.concatenate(
        [ei[:, p * SLAB:(p + 1) * SLAB].T for p in range(4)], axis=1)


def _node_features_t(ert, eit, fc_w, fc_b):
    bcol = fc_b[:, None]
    return pl.pallas_call(
        _fc_body,
        grid=(NCB,),
        in_specs=[
            pl.BlockSpec((EDIM, CB), lambda i: (0, i)),
            pl.BlockSpec((EDIM, CB), lambda i: (0, i)),
            pl.BlockSpec((HDIM, EDIM), lambda i: (0, 0)),
            pl.BlockSpec((HDIM, 1), lambda i: (0, 0)),
        ],
        out_specs=[
            pl.BlockSpec((HDIM, CB), lambda i: (0, i)),
            pl.BlockSpec((SLAB, W4), lambda i: (i, 0)),
            pl.BlockSpec((SLAB, W4), lambda i: (i, 0)),
        ],
        out_shape=[
            jax.ShapeDtypeStruct((HDIM, NUM_ENT), jnp.float32),
            jax.ShapeDtypeStruct((NPR, W4), jnp.float32),
            jax.ShapeDtypeStruct((NPR, W4), jnp.float32),
        ],
    )(ert, eit, fc_w, bcol)


def kernel(head, relation, tail, edge_index, edge_type,
           ent_real, ent_imag, rel_real, rel_imag, fc_w, fc_b):
    head = head.astype(jnp.int32)
    tail = tail.astype(jnp.int32)
    relation = relation.astype(jnp.int32)
    rel_tab = jnp.concatenate([
        rel_real[0], rel_imag[0], rel_real[1], rel_imag[1]])
    ert = ent_real.T
    eit = ent_imag.T
    nft, er4, ei4 = _node_features_t(ert, eit, fc_w, fc_b)
    score = _score_sc(head, relation, tail, er4, ei4, rel_tab)
    return (score, nft.T)


# speedup vs baseline: 1.5683x; 1.5683x over previous
import jax
import jax.numpy as jnp
from jax import lax
from jax.experimental import pallas as pl
from jax.experimental.pallas import tpu as pltpu
from jax.experimental.pallas import tpu_sc as plsc

NUM_ENT = 1000000
EDIM = 32
HDIM = 32
B = 16384

CB = 16384
SLAB = CB // 4
NCB = -(-NUM_ENT // CB)
NPR = NCB * SLAB
W4 = 128

NC = 2
NS = 16
L = 16
NW = NC * NS
BPW = B // NW
CH = 128
NCHUNK = BPW // CH


def _score_body(head_hbm, rel_hbm, tail_hbm, er_hbm, ei_hbm, rel_tab_hbm,
                out_hbm,
                hidx, tidx, relv, hgrp, tgrp, pkv, hr, hi, tr, ti, rtab, sco,
                sem):
    wid = lax.axis_index("s") * NC + lax.axis_index("c")
    base = wid * BPW
    pltpu.sync_copy(head_hbm.at[pl.ds(base, BPW)], hidx)
    pltpu.sync_copy(tail_hbm.at[pl.ds(base, BPW)], tidx)
    pltpu.sync_copy(rel_hbm.at[pl.ds(base, BPW)], relv)
    pltpu.sync_copy(rel_tab_hbm, rtab)

    def mkgrp(s, _):
        hc = hidx[pl.ds(s * L, L)]
        tc = tidx[pl.ds(s * L, L)]
        rc = relv[pl.ds(s * L, L)]
        hgrp[pl.ds(s * L, L)] = (
            lax.shift_left(lax.shift_right_logical(hc, 14), 12)
            | (hc & 4095))
        tgrp[pl.ds(s * L, L)] = (
            lax.shift_left(lax.shift_right_logical(tc, 14), 12)
            | (tc & 4095))
        pkv[pl.ds(s * L, L)] = (
            (lax.shift_right_logical(hc, 12) & 3)
            | lax.shift_left(lax.shift_right_logical(tc, 12) & 3, 2)
            | lax.shift_left(rc, 4))
        return _
    lax.fori_loop(0, BPW // L, mkgrp, 0)

    rr0 = [rtab[pl.ds(k * L, L)] for k in range(2)]
    ri0 = [rtab[pl.ds(EDIM + k * L, L)] for k in range(2)]
    rr1 = [rtab[pl.ds(2 * EDIM + k * L, L)] for k in range(2)]
    ri1 = [rtab[pl.ds(3 * EDIM + k * L, L)] for k in range(2)]
    lane = lax.broadcasted_iota(jnp.int32, (L,), 0)

    _gd = lax.GatherDimensionNumbers(
        offset_dims=(), collapsed_slice_dims=(0,), start_index_map=(0,))

    def vperm(v, idx):
        return lax.gather(v, idx[:, None], _gd, (1,),
                          mode=lax.GatherScatterMode.PROMISE_IN_BOUNDS)

    perm_idx = [lane ^ sh for sh in (8, 4, 2, 1)]

    def lane_sum(v):
        for idx in perm_idx:
            v = v + vperm(v, idx)
        return v

    jconst = [jnp.full((L,), j, jnp.int32) for j in range(L)]

    for c in range(NCHUNK):
        s = c * CH
        cps = [
            pltpu.async_copy(er_hbm.at[hgrp.at[pl.ds(s, CH)]], hr, sem),
            pltpu.async_copy(ei_hbm.at[hgrp.at[pl.ds(s, CH)]], hi, sem),
            pltpu.async_copy(er_hbm.at[tgrp.at[pl.ds(s, CH)]], tr, sem),
            pltpu.async_copy(ei_hbm.at[tgrp.at[pl.ds(s, CH)]], ti, sem),
        ]
        for cp in cps:
            cp.wait()

        def group(g, carry):
            pkc = pkv[pl.ds(s + g * L, L)]
            res = jnp.zeros((L,), jnp.float32)
            for j in range(L):
                r0 = g * L + j
                w = pkc[j]
                hoff = (w & 3) * EDIM
                toff = ((w >> 2) & 3) * EDIM
                rsel = w >> 4
                acc = jnp.zeros((L,), jnp.float32)
                for k in range(2):
                    hrk = hr[r0, pl.ds(hoff + k * L, L)]
                    hik = hi[r0, pl.ds(hoff + k * L, L)]
                    trk = tr[r0, pl.ds(toff + k * L, L)]
                    tik = ti[r0, pl.ds(toff + k * L, L)]
                    rrk = jnp.where(rsel == 0, rr0[k], rr1[k])
                    rik = jnp.where(rsel == 0, ri0[k], ri1[k])
                    u = hrk * rrk - hik * rik
                    v = hik * rrk + hrk * rik
                    acc = acc + trk * u + tik * v
                ssum = lane_sum(acc)
                res = jnp.where(lane == j, ssum, res)
            sco[pl.ds(s + g * L, L)] = res
            return carry

        lax.fori_loop(0, CH // L, group, 0)

    pltpu.sync_copy(sco, out_hbm.at[pl.ds(base, BPW)])


def _score_sc(head, relation, tail, er4, ei4, rel_tab):
    mesh = plsc.VectorSubcoreMesh(core_axis_name="c", subcore_axis_name="s",
                                  num_cores=NC, num_subcores=NS)
    fn = pl.kernel(
        _score_body,
        out_type=jax.ShapeDtypeStruct((B,), jnp.float32),
        mesh=mesh,
        scratch_types=[
            pltpu.VMEM((BPW,), jnp.int32),
            pltpu.VMEM((BPW,), jnp.int32),
            pltpu.VMEM((BPW,), jnp.int32),
            pltpu.VMEM((BPW,), jnp.int32),
            pltpu.VMEM((BPW,), jnp.int32),
            pltpu.VMEM((BPW,), jnp.int32),
            pltpu.VMEM((CH, W4), jnp.float32),
            pltpu.VMEM((CH, W4), jnp.float32),
            pltpu.VMEM((CH, W4), jnp.float32),
            pltpu.VMEM((CH, W4), jnp.float32),
            pltpu.VMEM((4 * EDIM,), jnp.float32),
            pltpu.VMEM((BPW,), jnp.float32),
            pltpu.SemaphoreType.DMA,
        ],
    )
    return fn(head, relation, tail, er4, ei4, rel_tab)


_DN = (((0,), (0,)), ((), ()))


def _fc_body(ert_ref, eit_ref, w_ref, b_ref, sel_ref, nft_ref, er4_ref,
             ei4_ref):
    er = ert_ref[...]
    ei = eit_ref[...]
    x = er + ei
    y = jnp.dot(w_ref[...], x, preferred_element_type=jnp.float32)
    nft_ref[...] = jnp.maximum(y + b_ref[...], 0.0)
    acc_r = None
    acc_i = None
    for p in range(4):
        sl = sel_ref[p]
        tr = lax.dot_general(er[:, p * SLAB:(p + 1) * SLAB], sl, _DN,
                             preferred_element_type=jnp.float32)
        ti = lax.dot_general(ei[:, p * SLAB:(p + 1) * SLAB], sl, _DN,
                             preferred_element_type=jnp.float32)
        acc_r = tr if acc_r is None else acc_r + tr
        acc_i = ti if acc_i is None else acc_i + ti
    er4_ref[...] = acc_r
    ei4_ref[...] = acc_i


def _node_features_t(ert, eit, fc_w, fc_b):
    bcol = fc_b[:, None]
    eye = jnp.eye(EDIM, dtype=jnp.float32)
    sel = jnp.stack([
        jnp.pad(eye, ((0, 0), (p * EDIM, W4 - (p + 1) * EDIM)))
        for p in range(4)])
    return pl.pallas_call(
        _fc_body,
        grid=(NCB,),
        in_specs=[
            pl.BlockSpec((EDIM, CB), lambda i: (0, i)),
            pl.BlockSpec((EDIM, CB), lambda i: (0, i)),
            pl.BlockSpec((HDIM, EDIM), lambda i: (0, 0)),
            pl.BlockSpec((HDIM, 1), lambda i: (0, 0)),
            pl.BlockSpec((4, EDIM, W4), lambda i: (0, 0, 0)),
        ],
        out_specs=[
            pl.BlockSpec((HDIM, CB), lambda i: (0, i)),
            pl.BlockSpec((SLAB, W4), lambda i: (i, 0)),
            pl.BlockSpec((SLAB, W4), lambda i: (i, 0)),
        ],
        out_shape=[
            jax.ShapeDtypeStruct((HDIM, NUM_ENT), jnp.float32),
            jax.ShapeDtypeStruct((NPR, W4), jnp.float32),
            jax.ShapeDtypeStruct((NPR, W4), jnp.float32),
        ],
    )(ert, eit, fc_w, bcol, sel)


def kernel(head, relation, tail, edge_index, edge_type,
           ent_real, ent_imag, rel_real, rel_imag, fc_w, fc_b):
    head = head.astype(jnp.int32)
    tail = tail.astype(jnp.int32)
    relation = relation.astype(jnp.int32)
    rel_tab = jnp.concatenate([
        rel_real[0], rel_imag[0], rel_real[1], rel_imag[1]])
    ert = ent_real.T
    eit = ent_imag.T
    nft, er4, ei4 = _node_features_t(ert, eit, fc_w, fc_b)
    score = _score_sc(head, relation, tail, er4, ei4, rel_tab)
    return (score, nft.T)


# speedup vs baseline: 2.1053x; 1.3424x over previous
import jax
import jax.numpy as jnp
from jax import lax
from jax.experimental import pallas as pl
from jax.experimental.pallas import tpu as pltpu
from jax.experimental.pallas import tpu_sc as plsc

NUM_ENT = 1000000
EDIM = 32
HDIM = 32
B = 16384

CB = 16384
SLAB = CB // 4
NCB = -(-NUM_ENT // CB)
NPR = NCB * SLAB
W4 = 128

NC = 2
NS = 16
L = 16
NW = NC * NS
BPW = B // NW
CH = 128
NCHUNK = BPW // CH


def _score_body(head_hbm, rel_hbm, tail_hbm, er_hbm, ei_hbm, rel_tab_hbm,
                out_hbm,
                hidx, tidx, relv, hgrp, tgrp, pkv, hr, hi, tr, ti, rtab, sco,
                sem):
    wid = lax.axis_index("s") * NC + lax.axis_index("c")
    base = wid * BPW
    pltpu.sync_copy(head_hbm.at[pl.ds(base, BPW)], hidx)
    pltpu.sync_copy(tail_hbm.at[pl.ds(base, BPW)], tidx)
    pltpu.sync_copy(rel_hbm.at[pl.ds(base, BPW)], relv)
    pltpu.sync_copy(rel_tab_hbm, rtab)

    def mkgrp(s, _):
        hc = hidx[pl.ds(s * L, L)]
        tc = tidx[pl.ds(s * L, L)]
        rc = relv[pl.ds(s * L, L)]
        hgrp[pl.ds(s * L, L)] = (
            lax.shift_left(lax.shift_right_logical(hc, 14), 12)
            | (hc & 4095))
        tgrp[pl.ds(s * L, L)] = (
            lax.shift_left(lax.shift_right_logical(tc, 14), 12)
            | (tc & 4095))
        pkv[pl.ds(s * L, L)] = (
            (lax.shift_right_logical(hc, 12) & 3)
            | lax.shift_left(lax.shift_right_logical(tc, 12) & 3, 2)
            | lax.shift_left(rc, 4))
        return _
    lax.fori_loop(0, BPW // L, mkgrp, 0)

    rr0 = [rtab[pl.ds(k * L, L)] for k in range(2)]
    ri0 = [rtab[pl.ds(EDIM + k * L, L)] for k in range(2)]
    rr1 = [rtab[pl.ds(2 * EDIM + k * L, L)] for k in range(2)]
    ri1 = [rtab[pl.ds(3 * EDIM + k * L, L)] for k in range(2)]
    lane = lax.broadcasted_iota(jnp.int32, (L,), 0)

    _gd = lax.GatherDimensionNumbers(
        offset_dims=(), collapsed_slice_dims=(0,), start_index_map=(0,))

    def vperm(v, idx):
        return lax.gather(v, idx[:, None], _gd, (1,),
                          mode=lax.GatherScatterMode.PROMISE_IN_BOUNDS)

    perm_idx = [lane ^ sh for sh in (8, 4, 2, 1)]

    def lane_sum(v):
        for idx in perm_idx:
            v = v + vperm(v, idx)
        return v

    jconst = [jnp.full((L,), j, jnp.int32) for j in range(L)]

    for c in range(NCHUNK):
        s = c * CH
        cps = [
            pltpu.async_copy(er_hbm.at[hgrp.at[pl.ds(s, CH)]], hr, sem),
            pltpu.async_copy(ei_hbm.at[hgrp.at[pl.ds(s, CH)]], hi, sem),
            pltpu.async_copy(er_hbm.at[tgrp.at[pl.ds(s, CH)]], tr, sem),
            pltpu.async_copy(ei_hbm.at[tgrp.at[pl.ds(s, CH)]], ti, sem),
        ]
        for cp in cps:
            cp.wait()

        def group(g, carry):
            pkc = pkv[pl.ds(s + g * L, L)]
            res = jnp.zeros((L,), jnp.float32)
            for j in range(L):
                r0 = g * L + j
                w = pkc[j]
                hoff = (w & 3) * EDIM
                toff = ((w >> 2) & 3) * EDIM
                rsel = w >> 4
                acc = jnp.zeros((L,), jnp.float32)
                for k in range(2):
                    hrk = hr[r0, pl.ds(hoff + k * L, L)]
                    hik = hi[r0, pl.ds(hoff + k * L, L)]
                    trk = tr[r0, pl.ds(toff + k * L, L)]
                    tik = ti[r0, pl.ds(toff + k * L, L)]
                    rrk = jnp.where(rsel == 0, rr0[k], rr1[k])
                    rik = jnp.where(rsel == 0, ri0[k], ri1[k])
                    u = hrk * rrk - hik * rik
                    v = hik * rrk + hrk * rik
                    acc = acc + trk * u + tik * v
                ssum = lane_sum(acc)
                res = jnp.where(lane == j, ssum, res)
            sco[pl.ds(s + g * L, L)] = res
            return carry

        lax.fori_loop(0, CH // L, group, 0)

    pltpu.sync_copy(sco, out_hbm.at[pl.ds(base, BPW)])


def _score_sc(head, relation, tail, er4, ei4, rel_tab):
    mesh = plsc.VectorSubcoreMesh(core_axis_name="c", subcore_axis_name="s",
                                  num_cores=NC, num_subcores=NS)
    fn = pl.kernel(
        _score_body,
        out_type=jax.ShapeDtypeStruct((B,), jnp.float32),
        mesh=mesh,
        scratch_types=[
            pltpu.VMEM((BPW,), jnp.int32),
            pltpu.VMEM((BPW,), jnp.int32),
            pltpu.VMEM((BPW,), jnp.int32),
            pltpu.VMEM((BPW,), jnp.int32),
            pltpu.VMEM((BPW,), jnp.int32),
            pltpu.VMEM((BPW,), jnp.int32),
            pltpu.VMEM((CH, W4), jnp.float32),
            pltpu.VMEM((CH, W4), jnp.float32),
            pltpu.VMEM((CH, W4), jnp.float32),
            pltpu.VMEM((CH, W4), jnp.float32),
            pltpu.VMEM((4 * EDIM,), jnp.float32),
            pltpu.VMEM((BPW,), jnp.float32),
            pltpu.SemaphoreType.DMA,
        ],
    )
    return fn(head, relation, tail, er4, ei4, rel_tab)


_DN = (((0,), (0,)), ((), ()))


def _fc_body(ert_ref, eit_ref, w_ref, b_ref, eye_ref, nft_ref, er4_ref,
             ei4_ref):
    er = ert_ref[...]
    ei = eit_ref[...]
    x = er + ei
    y = jnp.dot(w_ref[...], x, preferred_element_type=jnp.float32)
    nft_ref[...] = jnp.maximum(y + b_ref[...], 0.0)
    big_r = jnp.concatenate(
        [er[:, p * SLAB:(p + 1) * SLAB] for p in range(4)], axis=0)
    big_i = jnp.concatenate(
        [ei[:, p * SLAB:(p + 1) * SLAB] for p in range(4)], axis=0)
    er4_ref[...] = lax.dot_general(big_r, eye_ref[...], _DN,
                                   preferred_element_type=jnp.float32)
    ei4_ref[...] = lax.dot_general(big_i, eye_ref[...], _DN,
                                   preferred_element_type=jnp.float32)


def _node_features_t(ert, eit, fc_w, fc_b):
    bcol = fc_b[:, None]
    eye = jnp.eye(W4, dtype=jnp.float32)
    return pl.pallas_call(
        _fc_body,
        grid=(NCB,),
        in_specs=[
            pl.BlockSpec((EDIM, CB), lambda i: (0, i)),
            pl.BlockSpec((EDIM, CB), lambda i: (0, i)),
            pl.BlockSpec((HDIM, EDIM), lambda i: (0, 0)),
            pl.BlockSpec((HDIM, 1), lambda i: (0, 0)),
            pl.BlockSpec((W4, W4), lambda i: (0, 0)),
        ],
        out_specs=[
            pl.BlockSpec((HDIM, CB), lambda i: (0, i)),
            pl.BlockSpec((SLAB, W4), lambda i: (i, 0)),
            pl.BlockSpec((SLAB, W4), lambda i: (i, 0)),
        ],
        out_shape=[
            jax.ShapeDtypeStruct((HDIM, NUM_ENT), jnp.float32),
            jax.ShapeDtypeStruct((NPR, W4), jnp.float32),
            jax.ShapeDtypeStruct((NPR, W4), jnp.float32),
        ],
    )(ert, eit, fc_w, bcol, eye)


def kernel(head, relation, tail, edge_index, edge_type,
           ent_real, ent_imag, rel_real, rel_imag, fc_w, fc_b):
    head = head.astype(jnp.int32)
    tail = tail.astype(jnp.int32)
    relation = relation.astype(jnp.int32)
    rel_tab = jnp.concatenate([
        rel_real[0], rel_imag[0], rel_real[1], rel_imag[1]])
    ert = ent_real.T
    eit = ent_imag.T
    nft, er4, ei4 = _node_features_t(ert, eit, fc_w, fc_b)
    score = _score_sc(head, relation, tail, er4, ei4, rel_tab)
    return (score, nft.T)
